# Initial kernel scaffold; baseline (speedup 1.0000x reference)
import jax
import jax.numpy as jnp
from jax.experimental import pallas as pl


def kernel(X, G, W, b):
    raise NotImplementedError("write your pallas kernel here")


# speedup vs baseline: 2.1799x; 2.1799x over previous
import functools

import jax
import jax.numpy as jnp
from jax import lax
from jax.experimental import pallas as pl
from jax.experimental.pallas import tpu as pltpu

N = 8192
E = 8
D_IN = 1024
D_OUT = 1024
TM = 256


def _moe_tile(g_ref, x_ref, wt_ref, b_ref, out_ref):
    g = g_ref[...]

    neg_inf = jnp.float32(-jnp.inf)
    m1 = jnp.full((TM, 1), neg_inf, jnp.float32)
    i1 = jnp.zeros((TM, 1), jnp.int32)
    for e in range(E):
        ge = g[:, e : e + 1]
        better = ge > m1
        m1 = jnp.where(better, ge, m1)
        i1 = jnp.where(better, e, i1)
    m2 = jnp.full((TM, 1), neg_inf, jnp.float32)
    i2 = jnp.zeros((TM, 1), jnp.int32)
    for e in range(E):
        ge = jnp.where(i1 == e, neg_inf, g[:, e : e + 1])
        better = ge > m2
        m2 = jnp.where(better, ge, m2)
        i2 = jnp.where(better, e, i2)
    p1 = 1.0 / (1.0 + jnp.exp(m2 - m1))
    p2 = 1.0 - p1

    x = x_ref[...]
    acc = jnp.zeros((TM, D_OUT), jnp.float32)
    for e in range(E):
        se = jnp.where(i1 == e, p1, 0.0) + jnp.where(i2 == e, p2, 0.0)
        ye = lax.dot_general(
            x,
            wt_ref[e],
            (((1,), (0,)), ((), ())),
            preferred_element_type=jnp.float32,
        )
        acc += se * (ye + b_ref[e : e + 1, :])
    out_ref[...] = acc


@jax.jit
def kernel(X, G, W, b):
    Xb = X.astype(jnp.bfloat16)
    Wt = jnp.swapaxes(W, 1, 2).astype(jnp.bfloat16)
    grid = (N // TM,)
    return pl.pallas_call(
        _moe_tile,
        grid=grid,
        in_specs=[
            pl.BlockSpec((TM, E), lambda i: (i, 0)),
            pl.BlockSpec((TM, D_IN), lambda i: (i, 0)),
            pl.BlockSpec((E, D_IN, D_OUT), lambda i: (0, 0, 0)),
            pl.BlockSpec((E, D_OUT), lambda i: (0, 0)),
        ],
        out_specs=pl.BlockSpec((TM, D_OUT), lambda i: (i, 0)),
        out_shape=jax.ShapeDtypeStruct((N, D_OUT), jnp.float32),
        compiler_params=pltpu.CompilerParams(
            dimension_semantics=("arbitrary",),
        ),
    )(G, Xb, Wt, b)


# speedup vs baseline: 2.1957x; 1.0073x over previous
import functools

import jax
import jax.numpy as jnp
from jax import lax
from jax.experimental import pallas as pl
from jax.experimental.pallas import tpu as pltpu

N = 8192
E = 8
D_IN = 1024
D_OUT = 1024
TM = 512


def _moe_tile(g_ref, x_ref, wt_ref, b_ref, out_ref):
    g = g_ref[...]

    neg_inf = jnp.float32(-jnp.inf)
    m1 = jnp.full((TM, 1), neg_inf, jnp.float32)
    i1 = jnp.zeros((TM, 1), jnp.int32)
    for e in range(E):
        ge = g[:, e : e + 1]
        better = ge > m1
        m1 = jnp.where(better, ge, m1)
        i1 = jnp.where(better, e, i1)
    m2 = jnp.full((TM, 1), neg_inf, jnp.float32)
    i2 = jnp.zeros((TM, 1), jnp.int32)
    for e in range(E):
        ge = jnp.where(i1 == e, neg_inf, g[:, e : e + 1])
        better = ge > m2
        m2 = jnp.where(better, ge, m2)
        i2 = jnp.where(better, e, i2)
    p1 = 1.0 / (1.0 + jnp.exp(m2 - m1))
    p2 = 1.0 - p1

    x = x_ref[...]
    acc = jnp.zeros((TM, D_OUT), jnp.float32)
    for e in range(E):
        se = jnp.where(i1 == e, p1, 0.0) + jnp.where(i2 == e, p2, 0.0)
        ye = lax.dot_general(
            x,
            wt_ref[e],
            (((1,), (0,)), ((), ())),
            preferred_element_type=jnp.float32,
        )
        acc += se * (ye + b_ref[e : e + 1, :])
    out_ref[...] = acc


@jax.jit
def kernel(X, G, W, b):
    Xb = X.astype(jnp.bfloat16)
    Wt = jnp.swapaxes(W, 1, 2).astype(jnp.bfloat16)
    grid = (N // TM,)
    return pl.pallas_call(
        _moe_tile,
        grid=grid,
        in_specs=[
            pl.BlockSpec((TM, E), lambda i: (i, 0)),
            pl.BlockSpec((TM, D_IN), lambda i: (i, 0)),
            pl.BlockSpec((E, D_IN, D_OUT), lambda i: (0, 0, 0)),
            pl.BlockSpec((E, D_OUT), lambda i: (0, 0)),
        ],
        out_specs=pl.BlockSpec((TM, D_OUT), lambda i: (i, 0)),
        out_shape=jax.ShapeDtypeStruct((N, D_OUT), jnp.float32),
        compiler_params=pltpu.CompilerParams(
            dimension_semantics=("arbitrary",),
        ),
    )(G, Xb, Wt, b)


# speedup vs baseline: 2.3118x; 1.0529x over previous
import functools

import jax
import jax.numpy as jnp
from jax import lax
from jax.experimental import pallas as pl
from jax.experimental.pallas import tpu as pltpu

N = 8192
E = 8
D_IN = 1024
D_OUT = 1024
TM = 512


def _moe_tile(g_ref, x_ref, wt_ref, b_ref, out_ref):
    g = g_ref[...]

    neg_inf = jnp.float32(-jnp.inf)
    m1 = jnp.full((TM, 1), neg_inf, jnp.float32)
    i1 = jnp.zeros((TM, 1), jnp.int32)
    for e in range(E):
        ge = g[:, e : e + 1]
        better = ge > m1
        m1 = jnp.where(better, ge, m1)
        i1 = jnp.where(better, e, i1)
    m2 = jnp.full((TM, 1), neg_inf, jnp.float32)
    i2 = jnp.zeros((TM, 1), jnp.int32)
    for e in range(E):
        ge = jnp.where(i1 == e, neg_inf, g[:, e : e + 1])
        better = ge > m2
        m2 = jnp.where(better, ge, m2)
        i2 = jnp.where(better, e, i2)
    p1 = 1.0 / (1.0 + jnp.exp(m2 - m1))
    p2 = 1.0 - p1

    x = x_ref[...]
    acc = jnp.zeros((TM, D_OUT), jnp.float32)
    for e in range(E):
        se = jnp.where(i1 == e, p1, 0.0) + jnp.where(i2 == e, p2, 0.0)
        ye = lax.dot_general(
            x,
            wt_ref[e],
            (((1,), (1,)), ((), ())),
            preferred_element_type=jnp.float32,
        )
        acc += se * (ye + b_ref[e : e + 1, :])
    out_ref[...] = acc


@jax.jit
def kernel(X, G, W, b):
    Xb = X.astype(jnp.bfloat16)
    Wt = W.astype(jnp.bfloat16)
    grid = (N // TM,)
    return pl.pallas_call(
        _moe_tile,
        grid=grid,
        in_specs=[
            pl.BlockSpec((TM, E), lambda i: (i, 0)),
            pl.BlockSpec((TM, D_IN), lambda i: (i, 0)),
            pl.BlockSpec((E, D_OUT, D_IN), lambda i: (0, 0, 0)),
            pl.BlockSpec((E, D_OUT), lambda i: (0, 0)),
        ],
        out_specs=pl.BlockSpec((TM, D_OUT), lambda i: (i, 0)),
        out_shape=jax.ShapeDtypeStruct((N, D_OUT), jnp.float32),
        compiler_params=pltpu.CompilerParams(
            dimension_semantics=("arbitrary",),
        ),
    )(G, Xb, Wt, b)
